# Initial kernel scaffold; baseline (speedup 1.0000x reference)
import jax
import jax.numpy as jnp
from jax.experimental import pallas as pl


def kernel(x, edge_index, edge_weights, W1, b1, W2, b2):
    raise NotImplementedError("write your pallas kernel here")


# speedup vs baseline: 19.5427x; 19.5427x over previous
import functools

import jax
import jax.numpy as jnp
from jax import lax
from jax.experimental import pallas as pl
from jax.experimental.pallas import tpu as pltpu
from jax.experimental.pallas import tpu_sc as plsc

N_NODES = 10000
NP = 10240
TILE_ROWS = NP // 16
NP_DEG = NP
NW = 32
PER_W = 10240
EP = NW * PER_W
K = 64
NWIN = PER_W // K

_MESH = None


def _mesh():
    return plsc.VectorSubcoreMesh(
        core_axis_name="c", subcore_axis_name="s", num_cores=2, num_subcores=16
    )


def _make_deg_kernel():
    @functools.partial(
        pl.kernel,
        out_type=jax.ShapeDtypeStruct((2, NP_DEG), jnp.float32),
        mesh=_mesh(),
        compiler_params=pltpu.CompilerParams(needs_layout_passes=False),
        scratch_types=[
            pltpu.VMEM((NWIN, K), jnp.int32),
            pltpu.VMEM((NWIN, K), jnp.float32),
            pltpu.VMEM((NP_DEG // 16,), jnp.float32),
            pltpu.VMEM_SHARED((NP_DEG,), jnp.float32),
            pltpu.SemaphoreType.DMA,
            pltpu.SemaphoreType.DMA,
        ],
    )
    def deg_kernel(dst_hbm, ew_hbm, out_hbm, dst_v, ew_v, z_v, acc_sh,
                   sem0, sem1):
        c = lax.axis_index("c")
        s = lax.axis_index("s")
        wid = c * 16 + s
        trows = NP_DEG // 16
        row0 = s * trows
        zeros16 = jnp.zeros((16,), jnp.float32)

        def zstore(i, _):
            z_v[pl.ds(i * 16, 16)] = zeros16
            return 0

        lax.fori_loop(0, trows // 16, zstore, 0)
        pltpu.sync_copy(dst_hbm.at[pl.ds(wid * NWIN, NWIN)], dst_v)
        pltpu.sync_copy(ew_hbm.at[pl.ds(wid * NWIN, NWIN)], ew_v)
        pltpu.sync_copy(z_v, acc_sh.at[pl.ds(row0, trows)])
        plsc.subcore_barrier()

        sems = (sem0, sem1)

        def pair(i2, _):
            for b in range(2):
                w = i2 * 2 + b
                pltpu.async_copy(ew_v.at[w], acc_sh.at[dst_v.at[w]],
                                 sems[b], add=True)

                @pl.when(w >= 1)
                def _wait():
                    pltpu.make_async_copy(
                        ew_v.at[0], acc_sh.at[dst_v.at[0]], sems[1 - b]
                    ).wait()

            return 0

        lax.fori_loop(0, NWIN // 2, pair, 0)
        pltpu.make_async_copy(ew_v.at[0], acc_sh.at[dst_v.at[0]], sems[1]).wait()
        plsc.subcore_barrier()
        pltpu.sync_copy(acc_sh.at[pl.ds(row0, trows)],
                        out_hbm.at[c, pl.ds(row0, trows)])

    return deg_kernel


def _make_edge_scatter(d: int):
    @functools.partial(
        pl.kernel,
        out_type=jax.ShapeDtypeStruct((2, NP, d), jnp.float32),
        mesh=_mesh(),
        compiler_params=pltpu.CompilerParams(needs_layout_passes=False),
        scratch_types=[
            pltpu.VMEM((3, K), jnp.int32),
            pltpu.VMEM((3, K), jnp.int32),
            pltpu.VMEM((K,), jnp.int32),
            pltpu.VMEM((K,), jnp.int32),
            pltpu.VMEM((K, d), jnp.float32),
            pltpu.VMEM((K, d), jnp.float32),
            pltpu.VMEM((K, d), jnp.float32),
            pltpu.VMEM((K, d), jnp.float32),
            pltpu.VMEM((40, d), jnp.float32),
            pltpu.VMEM_SHARED((NP, d), jnp.float32),
            pltpu.SemaphoreType.DMA,
            pltpu.SemaphoreType.DMA,
            pltpu.SemaphoreType.DMA,
            pltpu.SemaphoreType.DMA,
            pltpu.SemaphoreType.DMA,
            pltpu.SemaphoreType.DMA,
        ],
    )
    def scatter_kernel(y_hbm, pk_hbm, out_hbm,
                       pkb0, pkb1, dsc0, dsc1,
                       g0, g1, sb0, sb1, z_v, acc_sh,
                       isem0, isem1, gsem0, gsem1, ssem0, ssem1):
        c = lax.axis_index("c")
        s = lax.axis_index("s")
        wid = c * 16 + s
        gw0 = wid * NWIN
        row0 = s * TILE_ROWS
        zeros16 = jnp.zeros((16,), jnp.float32)
        pkb = (pkb0, pkb1)
        dsc = (dsc0, dsc1)
        gbuf = (g0, g1)
        sbuf = (sb0, sb1)
        isem = (isem0, isem1)
        gsem = (gsem0, gsem1)
        ssem = (ssem0, ssem1)

        def idx_start(w, b):
            pltpu.async_copy(pk_hbm.at[gw0 + w], pkb[b], isem[b])

        def idx_wait(b):
            pltpu.make_async_copy(pk_hbm.at[0], pkb[b], isem[b]).wait()

        def gather_start(b):
            pltpu.async_copy(y_hbm.at[pkb[b].at[0]], gbuf[b], gsem[b])

        def gather_wait(b):
            pltpu.make_async_copy(y_hbm.at[pkb[b].at[0]], gbuf[b],
                                  gsem[b]).wait()

        def scatter_start(b):
            pltpu.async_copy(sbuf[b], acc_sh.at[dsc[b]], ssem[b], add=True)

        def scatter_wait(b):
            pltpu.make_async_copy(sbuf[b], acc_sh.at[dsc[b]], ssem[b]).wait()

        def zstore(r, _):
            for j in range(d // 16):
                z_v[r, pl.ds(j * 16, 16)] = zeros16
            return 0

        lax.fori_loop(0, 40, zstore, 0)

        def zcopy(i, _):
            pltpu.sync_copy(z_v, acc_sh.at[pl.ds(row0 + i * 40, 40)])
            return 0

        lax.fori_loop(0, TILE_ROWS // 40, zcopy, 0)
        plsc.subcore_barrier()

        idx_start(0, 0)
        idx_start(1, 1)
        idx_wait(0)
        gather_start(0)
        idx_wait(1)
        gather_start(1)

        def visit(w, b):
            gather_wait(b)

            @pl.when(jnp.logical_and(w >= 1, w + 1 < NWIN))
            def _next_gather():
                idx_wait(1 - b)
                gather_start(1 - b)

            @pl.when(w >= 2)
            def _wait_scatter():
                scatter_wait(b)

            for t in range(K // 16):
                dsc[b][pl.ds(t * 16, 16)] = pkb[b][1, pl.ds(t * 16, 16)]

            def scale(g, _):
                base = g * 16
                chunk = plsc.bitcast(pkb[b][2, pl.ds(base, 16)], jnp.float32)
                for l in range(16):
                    e = base + l
                    wv = chunk[l]
                    for j in range(d // 16):
                        sbuf[b][e, pl.ds(j * 16, 16)] = \
                            gbuf[b][e, pl.ds(j * 16, 16)] * wv
                return 0

            lax.fori_loop(0, K // 16, scale, 0)
            scatter_start(b)

            @pl.when(w + 2 < NWIN)
            def _next_idx():
                idx_start(w + 2, b)

        def pair(i2, _):
            for b in range(2):
                visit(i2 * 2 + b, b)
            return 0

        lax.fori_loop(0, NWIN // 2, pair, 0)
        scatter_wait(0)
        scatter_wait(1)
        plsc.subcore_barrier()
        pltpu.sync_copy(acc_sh.at[pl.ds(row0, TILE_ROWS)],
                        out_hbm.at[c, pl.ds(row0, TILE_ROWS)])

    return scatter_kernel


_deg_kernel = None
_edge_scatter_128 = None


def _sc_kernels():
    global _deg_kernel, _edge_scatter_128
    if _deg_kernel is None:
        _deg_kernel = _make_deg_kernel()
        _edge_scatter_128 = _make_edge_scatter(128)
    return _deg_kernel, _edge_scatter_128


_R = 2048
_GRID = NP // _R


def _tc1_body(x_ref, w_ref, degb_ref, y_ref):
    xw = jnp.dot(x_ref[...], w_ref[...], preferred_element_type=jnp.float32)
    dinv = lax.rsqrt(degb_ref[...] + 1.0)
    y_ref[...] = xw * dinv


def _tc1(x, w1, degb):
    return pl.pallas_call(
        _tc1_body,
        out_shape=jax.ShapeDtypeStruct((NP, 128), jnp.float32),
        grid=(_GRID,),
        in_specs=[
            pl.BlockSpec((_R, 128), lambda i: (i, 0)),
            pl.BlockSpec((128, 128), lambda i: (0, 0)),
            pl.BlockSpec((_R, 128), lambda i: (i, 0)),
        ],
        out_specs=pl.BlockSpec((_R, 128), lambda i: (i, 0)),
    )(x, w1, degb)


def _tc2_body(p0_ref, p1_ref, y1_ref, degb_ref, w2_ref, b1_ref, y2_ref):
    dinv = lax.rsqrt(degb_ref[...] + 1.0)
    h = dinv * (p0_ref[...] + p1_ref[...] + y1_ref[...]) + b1_ref[...]
    h = jnp.maximum(h, 0.0)
    xw2 = jnp.dot(h, w2_ref[...], preferred_element_type=jnp.float32)
    y2p = xw2 * dinv[:, :64]
    y2_ref[...] = jnp.concatenate([y2p, jnp.zeros_like(y2p)], axis=1)


def _tc2(p0, p1, y1, degb, w2, b1):
    return pl.pallas_call(
        _tc2_body,
        out_shape=jax.ShapeDtypeStruct((NP, 128), jnp.float32),
        grid=(_GRID,),
        in_specs=[
            pl.BlockSpec((_R, 128), lambda i: (i, 0)),
            pl.BlockSpec((_R, 128), lambda i: (i, 0)),
            pl.BlockSpec((_R, 128), lambda i: (i, 0)),
            pl.BlockSpec((_R, 128), lambda i: (i, 0)),
            pl.BlockSpec((128, 64), lambda i: (0, 0)),
            pl.BlockSpec((1, 128), lambda i: (0, 0)),
        ],
        out_specs=pl.BlockSpec((_R, 128), lambda i: (i, 0)),
    )(p0, p1, y1, degb, w2, b1)


def _tc3_body(q0_ref, q1_ref, y2_ref, degb_ref, b2_ref, out_ref):
    dinv = lax.rsqrt(degb_ref[...] + 1.0)[:, :64]
    t = dinv * (q0_ref[...] + q1_ref[...] + y2_ref[...])[:, :64] + b2_ref[...]
    m = jnp.max(t, axis=1, keepdims=True)
    lse = jnp.log(jnp.sum(jnp.exp(t - m), axis=1, keepdims=True))
    out_ref[...] = t - m - lse


def _tc3(q0, q1, y2, degb, b2):
    return pl.pallas_call(
        _tc3_body,
        out_shape=jax.ShapeDtypeStruct((NP, 64), jnp.float32),
        grid=(_GRID,),
        in_specs=[
            pl.BlockSpec((_R, 128), lambda i: (i, 0)),
            pl.BlockSpec((_R, 128), lambda i: (i, 0)),
            pl.BlockSpec((_R, 128), lambda i: (i, 0)),
            pl.BlockSpec((_R, 128), lambda i: (i, 0)),
            pl.BlockSpec((1, 64), lambda i: (0, 0)),
        ],
        out_specs=pl.BlockSpec((_R, 64), lambda i: (i, 0)),
    )(q0, q1, y2, degb, b2)


def kernel(x, edge_index, edge_weights, W1, b1, W2, b2):
    n = x.shape[0]
    e = edge_weights.shape[0]
    src = edge_index[0].astype(jnp.int32)
    dst = edge_index[1].astype(jnp.int32)
    ew = edge_weights.astype(jnp.float32)

    pad = EP - e
    pad_idx = jnp.arange(pad, dtype=jnp.int32) % n
    src_p = jnp.concatenate([src, pad_idx])
    dst_p = jnp.concatenate([dst, pad_idx])
    ew_p = jnp.concatenate([ew, jnp.zeros((pad,), jnp.float32)])
    ew_bits = lax.bitcast_convert_type(ew_p, jnp.int32)
    pk = jnp.concatenate([src_p.reshape(EP // K, 1, K),
                          dst_p.reshape(EP // K, 1, K),
                          ew_bits.reshape(EP // K, 1, K)], axis=1)
    dst_w = dst_p.reshape(EP // K, K)
    ew_w = ew_p.reshape(EP // K, K)

    x_p = jnp.pad(x, ((0, NP - n), (0, 0)))

    deg_kernel, edge_scatter = _sc_kernels()
    degp = deg_kernel(dst_w, ew_w)
    degb = jnp.broadcast_to((degp[0] + degp[1])[:, None], (NP, 128))

    y1 = _tc1(x_p, W1, degb)
    p = edge_scatter(y1, pk)
    y2 = _tc2(p[0], p[1], y1, degb, W2, b1.reshape(1, -1))
    q = edge_scatter(y2, pk)
    out = _tc3(q[0], q[1], y2, degb, b2.reshape(1, -1))
    return out[:n]


# speedup vs baseline: 19.7688x; 1.0116x over previous
import functools

import jax
import jax.numpy as jnp
from jax import lax
from jax.experimental import pallas as pl
from jax.experimental.pallas import tpu as pltpu
from jax.experimental.pallas import tpu_sc as plsc

N_NODES = 10000
NP = 10240
TILE_ROWS = NP // 16
NP_DEG = NP
NW = 32
PER_W = 10240
EP = NW * PER_W
K = 64
NWIN = PER_W // K

_MESH = None


def _mesh():
    return plsc.VectorSubcoreMesh(
        core_axis_name="c", subcore_axis_name="s", num_cores=2, num_subcores=16
    )


def _make_deg_kernel():
    @functools.partial(
        pl.kernel,
        out_type=jax.ShapeDtypeStruct((2, NP_DEG), jnp.float32),
        mesh=_mesh(),
        compiler_params=pltpu.CompilerParams(needs_layout_passes=False),
        scratch_types=[
            pltpu.VMEM((NWIN, K), jnp.int32),
            pltpu.VMEM((NWIN, K), jnp.float32),
            pltpu.VMEM((NP_DEG // 16,), jnp.float32),
            pltpu.VMEM_SHARED((NP_DEG,), jnp.float32),
            pltpu.SemaphoreType.DMA,
            pltpu.SemaphoreType.DMA,
        ],
    )
    def deg_kernel(dst_hbm, ew_hbm, out_hbm, dst_v, ew_v, z_v, acc_sh,
                   sem0, sem1):
        c = lax.axis_index("c")
        s = lax.axis_index("s")
        wid = c * 16 + s
        trows = NP_DEG // 16
        row0 = s * trows
        zeros16 = jnp.zeros((16,), jnp.float32)

        def zstore(i, _):
            z_v[pl.ds(i * 16, 16)] = zeros16
            return 0

        lax.fori_loop(0, trows // 16, zstore, 0)
        pltpu.sync_copy(dst_hbm.at[pl.ds(wid * NWIN, NWIN)], dst_v)
        pltpu.sync_copy(ew_hbm.at[pl.ds(wid * NWIN, NWIN)], ew_v)
        pltpu.sync_copy(z_v, acc_sh.at[pl.ds(row0, trows)])
        plsc.subcore_barrier()

        sems = (sem0, sem1)

        def pair(i2, _):
            for b in range(2):
                w = i2 * 2 + b
                pltpu.async_copy(ew_v.at[w], acc_sh.at[dst_v.at[w]],
                                 sems[b], add=True)

                @pl.when(w >= 1)
                def _wait():
                    pltpu.make_async_copy(
                        ew_v.at[0], acc_sh.at[dst_v.at[0]], sems[1 - b]
                    ).wait()

            return 0

        lax.fori_loop(0, NWIN // 2, pair, 0)
        pltpu.make_async_copy(ew_v.at[0], acc_sh.at[dst_v.at[0]], sems[1]).wait()
        plsc.subcore_barrier()
        pltpu.sync_copy(acc_sh.at[pl.ds(row0, trows)],
                        out_hbm.at[c, pl.ds(row0, trows)])

    return deg_kernel


def _make_edge_scatter(d: int):
    @functools.partial(
        pl.kernel,
        out_type=jax.ShapeDtypeStruct((2, NP, d), jnp.float32),
        mesh=_mesh(),
        compiler_params=pltpu.CompilerParams(needs_layout_passes=False),
        scratch_types=[
            pltpu.VMEM((K,), jnp.int32),
            pltpu.VMEM((K,), jnp.int32),
            pltpu.VMEM((K,), jnp.int32),
            pltpu.VMEM((K,), jnp.int32),
            pltpu.VMEM((K,), jnp.float32),
            pltpu.VMEM((K,), jnp.float32),
            pltpu.VMEM((K,), jnp.int32),
            pltpu.VMEM((K,), jnp.int32),
            pltpu.VMEM((K, d), jnp.float32),
            pltpu.VMEM((K, d), jnp.float32),
            pltpu.VMEM((K, d), jnp.float32),
            pltpu.VMEM((K, d), jnp.float32),
            pltpu.VMEM((40, d), jnp.float32),
            pltpu.VMEM_SHARED((NP, d), jnp.float32),
            pltpu.SemaphoreType.DMA,
            pltpu.SemaphoreType.DMA,
            pltpu.SemaphoreType.DMA,
            pltpu.SemaphoreType.DMA,
            pltpu.SemaphoreType.DMA,
            pltpu.SemaphoreType.DMA,
        ],
    )
    def scatter_kernel(y_hbm, src_hbm, dst_hbm, ew_hbm, out_hbm,
                       si0, si1, di0, di1, ewb0, ewb1, dsc0, dsc1,
                       g0, g1, sb0, sb1, z_v, acc_sh,
                       isem0, isem1, gsem0, gsem1, ssem0, ssem1):
        c = lax.axis_index("c")
        s = lax.axis_index("s")
        wid = c * 16 + s
        gw0 = wid * NWIN
        row0 = s * TILE_ROWS
        zeros16 = jnp.zeros((16,), jnp.float32)
        si = (si0, si1)
        di = (di0, di1)
        ewb = (ewb0, ewb1)
        dsc = (dsc0, dsc1)
        gbuf = (g0, g1)
        sbuf = (sb0, sb1)
        isem = (isem0, isem1)
        gsem = (gsem0, gsem1)
        ssem = (ssem0, ssem1)

        def idx_start(w, b):
            pltpu.async_copy(src_hbm.at[gw0 + w], si[b], isem[b])
            pltpu.async_copy(dst_hbm.at[gw0 + w], di[b], isem[b])
            pltpu.async_copy(ew_hbm.at[gw0 + w], ewb[b], isem[b])

        def idx_wait(b):
            pltpu.make_async_copy(src_hbm.at[0], si[b], isem[b]).wait()
            pltpu.make_async_copy(dst_hbm.at[0], di[b], isem[b]).wait()
            pltpu.make_async_copy(ew_hbm.at[0], ewb[b], isem[b]).wait()

        def gather_start(b):
            pltpu.async_copy(y_hbm.at[si[b]], gbuf[b], gsem[b])

        def gather_wait(b):
            pltpu.make_async_copy(y_hbm.at[si[b]], gbuf[b], gsem[b]).wait()

        def scatter_start(b):
            pltpu.async_copy(sbuf[b], acc_sh.at[dsc[b]], ssem[b], add=True)

        def scatter_wait(b):
            pltpu.make_async_copy(sbuf[b], acc_sh.at[dsc[b]], ssem[b]).wait()

        def zstore(r, _):
            for j in range(d // 16):
                z_v[r, pl.ds(j * 16, 16)] = zeros16
            return 0

        lax.fori_loop(0, 40, zstore, 0)

        def zcopy(i, _):
            pltpu.sync_copy(z_v, acc_sh.at[pl.ds(row0 + i * 40, 40)])
            return 0

        lax.fori_loop(0, TILE_ROWS // 40, zcopy, 0)
        plsc.subcore_barrier()

        idx_start(0, 0)
        idx_start(1, 1)
        idx_wait(0)
        gather_start(0)
        idx_wait(1)
        gather_start(1)

        def visit(w, b):
            gather_wait(b)

            @pl.when(jnp.logical_and(w >= 1, w + 1 < NWIN))
            def _next_gather():
                idx_wait(1 - b)
                gather_start(1 - b)

            @pl.when(w >= 2)
            def _wait_scatter():
                scatter_wait(b)

            for t in range(K // 16):
                dsc[b][pl.ds(t * 16, 16)] = di[b][pl.ds(t * 16, 16)]

            def scale(g, _):
                base = g * 16
                chunk = ewb[b][pl.ds(base, 16)]
                for l in range(16):
                    e = base + l
                    wv = chunk[l]
                    for j in range(d // 16):
                        sbuf[b][e, pl.ds(j * 16, 16)] = \
                            gbuf[b][e, pl.ds(j * 16, 16)] * wv
                return 0

            lax.fori_loop(0, K // 16, scale, 0)
            scatter_start(b)

            @pl.when(w + 2 < NWIN)
            def _next_idx():
                idx_start(w + 2, b)

        def pair(i2, _):
            for b in range(2):
                visit(i2 * 2 + b, b)
            return 0

        lax.fori_loop(0, NWIN // 2, pair, 0)
        scatter_wait(0)
        scatter_wait(1)
        plsc.subcore_barrier()
        pltpu.sync_copy(acc_sh.at[pl.ds(row0, TILE_ROWS)],
                        out_hbm.at[c, pl.ds(row0, TILE_ROWS)])

    return scatter_kernel


_deg_kernel = None
_edge_scatter_128 = None


def _sc_kernels():
    global _deg_kernel, _edge_scatter_128
    if _deg_kernel is None:
        _deg_kernel = _make_deg_kernel()
        _edge_scatter_128 = _make_edge_scatter(128)
    return _deg_kernel, _edge_scatter_128


_R = 2048
_GRID = NP // _R


def _tc1_body(x_ref, w_ref, degb_ref, y_ref):
    xw = jnp.dot(x_ref[...], w_ref[...], preferred_element_type=jnp.float32)
    dinv = lax.rsqrt(degb_ref[...] + 1.0)
    y_ref[...] = xw * dinv


def _tc1(x, w1, degb):
    return pl.pallas_call(
        _tc1_body,
        out_shape=jax.ShapeDtypeStruct((NP, 128), jnp.float32),
        grid=(_GRID,),
        in_specs=[
            pl.BlockSpec((_R, 128), lambda i: (i, 0)),
            pl.BlockSpec((128, 128), lambda i: (0, 0)),
            pl.BlockSpec((_R, 128), lambda i: (i, 0)),
        ],
        out_specs=pl.BlockSpec((_R, 128), lambda i: (i, 0)),
    )(x, w1, degb)


def _tc2_body(p0_ref, p1_ref, y1_ref, degb_ref, w2_ref, b1_ref, y2_ref):
    dinv = lax.rsqrt(degb_ref[...] + 1.0)
    h = dinv * (p0_ref[...] + p1_ref[...] + y1_ref[...]) + b1_ref[...]
    h = jnp.maximum(h, 0.0)
    xw2 = jnp.dot(h, w2_ref[...], preferred_element_type=jnp.float32)
    y2p = xw2 * dinv[:, :64]
    y2_ref[...] = jnp.concatenate([y2p, jnp.zeros_like(y2p)], axis=1)


def _tc2(p0, p1, y1, degb, w2, b1):
    return pl.pallas_call(
        _tc2_body,
        out_shape=jax.ShapeDtypeStruct((NP, 128), jnp.float32),
        grid=(_GRID,),
        in_specs=[
            pl.BlockSpec((_R, 128), lambda i: (i, 0)),
            pl.BlockSpec((_R, 128), lambda i: (i, 0)),
            pl.BlockSpec((_R, 128), lambda i: (i, 0)),
            pl.BlockSpec((_R, 128), lambda i: (i, 0)),
            pl.BlockSpec((128, 64), lambda i: (0, 0)),
            pl.BlockSpec((1, 128), lambda i: (0, 0)),
        ],
        out_specs=pl.BlockSpec((_R, 128), lambda i: (i, 0)),
    )(p0, p1, y1, degb, w2, b1)


def _tc3_body(q0_ref, q1_ref, y2_ref, degb_ref, b2_ref, out_ref):
    dinv = lax.rsqrt(degb_ref[...] + 1.0)[:, :64]
    t = dinv * (q0_ref[...] + q1_ref[...] + y2_ref[...])[:, :64] + b2_ref[...]
    m = jnp.max(t, axis=1, keepdims=True)
    lse = jnp.log(jnp.sum(jnp.exp(t - m), axis=1, keepdims=True))
    out_ref[...] = t - m - lse


def _tc3(q0, q1, y2, degb, b2):
    return pl.pallas_call(
        _tc3_body,
        out_shape=jax.ShapeDtypeStruct((NP, 64), jnp.float32),
        grid=(_GRID,),
        in_specs=[
            pl.BlockSpec((_R, 128), lambda i: (i, 0)),
            pl.BlockSpec((_R, 128), lambda i: (i, 0)),
            pl.BlockSpec((_R, 128), lambda i: (i, 0)),
            pl.BlockSpec((_R, 128), lambda i: (i, 0)),
            pl.BlockSpec((1, 64), lambda i: (0, 0)),
        ],
        out_specs=pl.BlockSpec((_R, 64), lambda i: (i, 0)),
    )(q0, q1, y2, degb, b2)


def kernel(x, edge_index, edge_weights, W1, b1, W2, b2):
    n = x.shape[0]
    e = edge_weights.shape[0]
    src = edge_index[0].astype(jnp.int32)
    dst = edge_index[1].astype(jnp.int32)
    ew = edge_weights.astype(jnp.float32)

    pad = EP - e
    pad_idx = jnp.arange(pad, dtype=jnp.int32) % n
    src_p = jnp.concatenate([src, pad_idx]).reshape(EP // K, K)
    dst_p = jnp.concatenate([dst, pad_idx]).reshape(EP // K, K)
    ew_p = jnp.concatenate([ew, jnp.zeros((pad,), jnp.float32)]).reshape(EP // K, K)

    x_p = jnp.pad(x, ((0, NP - n), (0, 0)))

    deg_kernel, edge_scatter = _sc_kernels()
    degp = deg_kernel(dst_p, ew_p)
    degb = jnp.broadcast_to((degp[0] + degp[1])[:, None], (NP, 128))

    y1 = _tc1(x_p, W1, degb)
    p = edge_scatter(y1, src_p, dst_p, ew_p)
    y2 = _tc2(p[0], p[1], y1, degb, W2, b1.reshape(1, -1))
    q = edge_scatter(y2, src_p, dst_p, ew_p)
    out = _tc3(q[0], q[1], y2, degb, b2.reshape(1, -1))
    return out[:n]
